# baseline (speedup 1.0000x reference)
import jax
import jax.numpy as jnp
from jax.experimental import pallas as pl

_SEQ = 1729
_PATCH = 576


def _add_pe_block(x_ref, pe_ref, o_ref, *, rows_per_block: int):
    pid = pl.program_id(0)
    g = pid * rows_per_block + jax.lax.broadcasted_iota(
        jnp.int32, (rows_per_block, 1), 0
    )
    t = g % _SEQ
    pos = (t + (_PATCH - 1)) // _PATCH
    pe0 = pe_ref[0:1, :]
    pe1 = pe_ref[1:2, :]
    pe2 = pe_ref[2:3, :]
    pe3 = pe_ref[3:4, :]
    addend = jnp.where(
        pos <= 1,
        jnp.where(pos == 0, pe0, pe1),
        jnp.where(pos == 2, pe2, pe3),
    )
    o_ref[...] = x_ref[...] + addend


def kernel(x, pe):
    B, S, D = x.shape
    xf = x.reshape(B * S, D)
    rows = B * S
    rpb = 728
    grid = rows // rpb
    out = pl.pallas_call(
        lambda x_ref, pe_ref, o_ref: _add_pe_block(
            x_ref, pe_ref, o_ref, rows_per_block=rpb
        ),
        grid=(grid,),
        in_specs=[
            pl.BlockSpec((rpb, D), lambda i: (i, 0)),
            pl.BlockSpec((4, D), lambda i: (0, 0)),
        ],
        out_specs=pl.BlockSpec((rpb, D), lambda i: (i, 0)),
        out_shape=jax.ShapeDtypeStruct((rows, D), x.dtype),
    )(xf, pe)
    return out.reshape(B, S, D)


# speedup vs baseline: 1.4979x; 1.4979x over previous
import jax
import jax.numpy as jnp
from jax.experimental import pallas as pl

_SEQ = 1729
_PATCH = 576


def _add_pe_block(x_ref, pe_ref, o_ref, *, tok_per_block: int):
    j = pl.program_id(1)
    t = j * tok_per_block + jax.lax.broadcasted_iota(
        jnp.int32, (1, tok_per_block, 1), 1
    )
    pos = (t + (_PATCH - 1)) // _PATCH
    pe0 = pe_ref[0:1, :][None]
    pe1 = pe_ref[1:2, :][None]
    pe2 = pe_ref[2:3, :][None]
    pe3 = pe_ref[3:4, :][None]
    addend = jnp.where(
        pos <= 1,
        jnp.where(pos == 0, pe0, pe1),
        jnp.where(pos == 2, pe2, pe3),
    )
    o_ref[...] = x_ref[...] + addend


def kernel(x, pe):
    B, S, D = x.shape
    tpb = 256
    jblocks = pl.cdiv(S, tpb)
    out = pl.pallas_call(
        lambda x_ref, pe_ref, o_ref: _add_pe_block(
            x_ref, pe_ref, o_ref, tok_per_block=tpb
        ),
        grid=(B, jblocks),
        in_specs=[
            pl.BlockSpec((1, tpb, D), lambda b, j: (b, j, 0)),
            pl.BlockSpec((4, D), lambda b, j: (0, 0)),
        ],
        out_specs=pl.BlockSpec((1, tpb, D), lambda b, j: (b, j, 0)),
        out_shape=jax.ShapeDtypeStruct((B, S, D), x.dtype),
    )(x, pe)
    return out


# speedup vs baseline: 1.5300x; 1.0214x over previous
import jax
import jax.numpy as jnp
from jax import lax
from jax.experimental import pallas as pl
from jax.experimental.pallas import tpu as pltpu
from jax.experimental.pallas import tpu_sc as plsc

_SEQ = 1729
_PATCH = 576
_D = 768
_LANES = 16
_NVEC = _D // _LANES
_CHUNK = 96


def _pos(t):
    return (t + _PATCH - 1) // _PATCH


def _chunk_plan():
    plan = []
    for c in range(_SEQ // _CHUNK):
        t0 = c * _CHUNK
        splits = []
        start = 0
        while start < _CHUNK:
            row = _pos(t0 + start)
            end = start
            while end < _CHUNK and _pos(t0 + end) == row:
                end += 1
            splits.append((start, end - start, row))
            start = end
        plan.append((t0, _CHUNK, splits))
    rem = _SEQ % _CHUNK
    if rem:
        t0 = _SEQ - rem
        plan.append((t0, rem, [(0, rem, _pos(t0))]))
    return plan


_PLAN = _chunk_plan()


def _sc_body(x_hbm, pe_hbm, out_hbm, pe_v, buf):
    nc = 2
    wid = lax.axis_index("s") * nc + lax.axis_index("c")

    pltpu.sync_copy(pe_hbm, pe_v)
    pe_rows = [
        [pe_v[row, pl.ds(k * _LANES, _LANES)] for k in range(_NVEC)]
        for row in range(4)
    ]

    def add_range(r0, cnt, vals):
        if cnt == 1:
            for k in range(_NVEC):
                buf[r0, pl.ds(k * _LANES, _LANES)] += vals[k]
            return

        def body(r, carry):
            for k in range(_NVEC):
                buf[r, pl.ds(k * _LANES, _LANES)] += vals[k]
            return carry

        lax.fori_loop(r0, r0 + cnt, body, jnp.int32(0))

    for t0, n, splits in _PLAN:
        pltpu.sync_copy(x_hbm.at[wid, pl.ds(t0, n)], buf.at[pl.ds(0, n)])
        for r0, cnt, row in splits:
            add_range(r0, cnt, pe_rows[row])
        pltpu.sync_copy(buf.at[pl.ds(0, n)], out_hbm.at[wid, pl.ds(t0, n)])


def kernel(x, pe):
    B, S, D = x.shape
    mesh = plsc.VectorSubcoreMesh(core_axis_name="c", subcore_axis_name="s")
    sc_add = pl.kernel(
        _sc_body,
        out_type=jax.ShapeDtypeStruct((B, S, D), x.dtype),
        mesh=mesh,
        scratch_types=[
            pltpu.VMEM((4, D), jnp.float32),
            pltpu.VMEM((_CHUNK, D), jnp.float32),
        ],
    )
    return sc_add(x, pe)


# speedup vs baseline: 1.6977x; 1.1096x over previous
import jax
import jax.numpy as jnp
from jax import lax
from jax.experimental import pallas as pl
from jax.experimental.pallas import tpu as pltpu
from jax.experimental.pallas import tpu_sc as plsc

_SEQ = 1729
_PATCH = 576
_D = 768
_LANES = 16
_NVEC = _D // _LANES
_CHUNK = 80


def _pos(t):
    return (t + _PATCH - 1) // _PATCH


def _chunk_plan():
    plan = []
    t0 = 0
    while t0 < _SEQ:
        n = min(_CHUNK, _SEQ - t0)
        n = n if n % 8 == 0 else (n - n % 8 or n)
        splits = []
        start = 0
        while start < n:
            row = _pos(t0 + start)
            end = start
            while end < n and _pos(t0 + end) == row:
                end += 1
            splits.append((start, end - start, row))
            start = end
        plan.append((t0, n, splits))
        t0 += n
    return plan


_PLAN = _chunk_plan()


def _sc_body(x_hbm, pe_hbm, out_hbm, pe_v, buf0, buf1, si0, si1, so0, so1):
    nc = 2
    wid = lax.axis_index("s") * nc + lax.axis_index("c")

    pltpu.sync_copy(pe_hbm, pe_v)
    pe_rows = [
        [pe_v[row, pl.ds(k * _LANES, _LANES)] for k in range(_NVEC)]
        for row in range(4)
    ]

    bufs = (buf0, buf1)
    sin = (si0, si1)
    sout = (so0, so1)

    def add_range(b, r0, cnt, vals):
        buf = bufs[b]
        if cnt == 1:
            for k in range(_NVEC):
                buf[r0, pl.ds(k * _LANES, _LANES)] += vals[k]
            return

        def body(r, carry):
            for k in range(_NVEC):
                buf[r, pl.ds(k * _LANES, _LANES)] += vals[k]
            return carry

        lax.fori_loop(r0, r0 + cnt, body, jnp.int32(0))

    def start_in(i):
        t0, n, _ = _PLAN[i]
        b = i % 2
        return pltpu.make_async_copy(
            x_hbm.at[wid, pl.ds(t0, n)], bufs[b].at[pl.ds(0, n)], sin[b]
        )

    def start_out(i):
        t0, n, _ = _PLAN[i]
        b = i % 2
        return pltpu.make_async_copy(
            bufs[b].at[pl.ds(0, n)], out_hbm.at[wid, pl.ds(t0, n)], sout[b]
        )

    nchunks = len(_PLAN)
    din = start_in(0)
    din.start()
    dout_prev = None
    dout_prev2 = None
    for i in range(nchunks):
        b = i % 2
        din.wait()
        if i + 1 < nchunks:
            if dout_prev is not None:
                dout_prev.wait()
            din = start_in(i + 1)
            din.start()
        for r0, cnt, row in _PLAN[i][2]:
            add_range(b, r0, cnt, pe_rows[row])
        d = start_out(i)
        d.start()
        dout_prev2 = dout_prev
        dout_prev = d
    if dout_prev2 is not None:
        dout_prev2.wait()
    if dout_prev is not None:
        dout_prev.wait()


def kernel(x, pe):
    B, S, D = x.shape
    mesh = plsc.VectorSubcoreMesh(core_axis_name="c", subcore_axis_name="s")
    sc_add = pl.kernel(
        _sc_body,
        out_type=jax.ShapeDtypeStruct((B, S, D), x.dtype),
        mesh=mesh,
        scratch_types=[
            pltpu.VMEM((4, D), jnp.float32),
            pltpu.VMEM((_CHUNK, D), jnp.float32),
            pltpu.VMEM((_CHUNK, D), jnp.float32),
            pltpu.SemaphoreType.DMA,
            pltpu.SemaphoreType.DMA,
            pltpu.SemaphoreType.DMA,
            pltpu.SemaphoreType.DMA,
        ],
    )
    return sc_add(x, pe)


# speedup vs baseline: 4.6323x; 2.7286x over previous
import jax
import jax.numpy as jnp
from jax import lax
from jax.experimental import pallas as pl
from jax.experimental.pallas import tpu as pltpu
from jax.experimental.pallas import tpu_sc as plsc

_SEQ = 1729
_PATCH = 576
_B = 32
_D = 768
_LANES = 16
_NVEC = _D // _LANES
_NW = 32
_NTOK = 55
_NBUF = 4


def _sc_body(xt_hbm, pe_hbm, out_hbm, pe_v, bufs, sins, souts):
    nc = 2
    wid = lax.axis_index("s") * nc + lax.axis_index("c")
    base = wid * (_NTOK - 1)

    pltpu.sync_copy(pe_hbm, pe_v)

    def in_start(c, b):
        pltpu.make_async_copy(
            xt_hbm.at[pl.ds(base + c, 1)], bufs[b], sins[b]
        ).start()

    def in_wait(b):
        pltpu.make_async_copy(
            xt_hbm.at[pl.ds(0, 1)], bufs[b], sins[b]
        ).wait()

    def out_start(c, b):
        pltpu.make_async_copy(
            bufs[b], out_hbm.at[pl.ds(base + c, 1)], souts[b]
        ).start()

    def out_wait(b):
        pltpu.make_async_copy(
            bufs[b], out_hbm.at[pl.ds(0, 1)], souts[b]
        ).wait()

    def compute(c, b):
        t = base + c
        pos = (t + _PATCH - 1) // _PATCH
        vals = [pe_v[pos, pl.ds(k * _LANES, _LANES)] for k in range(_NVEC)]
        buf = bufs[b]

        def body(j, carry):
            for k in range(_NVEC):
                buf[0, j, pl.ds(k * _LANES, _LANES)] += vals[k]
            return carry

        lax.fori_loop(0, _B, body, jnp.int32(0))

    def step(j, par, c2_valid, c2_wait):
        b2 = (par + 2) % _NBUF
        if c2_wait:
            out_wait(b2)
        if c2_valid:
            in_start(j + 2, b2)
        b = par % _NBUF
        in_wait(b)
        compute(j, b)
        out_start(j, b)

    in_start(0, 0)
    in_start(1, 1)
    step(0, 0, True, False)
    step(1, 1, True, False)

    def group(m, carry):
        j0 = 2 + 4 * m
        for u in range(4):
            step(j0 + u, 2 + u, True, True)
        return carry

    lax.fori_loop(0, 12, group, jnp.int32(0))

    for j in range(50, _NTOK):
        c2 = j + 2
        step(j, j % _NBUF, c2 < _NTOK, c2 < _NTOK)
    for j in range(_NTOK - _NBUF, _NTOK):
        out_wait(j % _NBUF)


def kernel(x, pe):
    B, S, D = x.shape
    xt = jnp.transpose(x, (1, 0, 2))
    mesh = plsc.VectorSubcoreMesh(core_axis_name="c", subcore_axis_name="s")
    sc_add = pl.kernel(
        _sc_body,
        out_type=jax.ShapeDtypeStruct((S, B, D), x.dtype),
        mesh=mesh,
        scratch_types=[
            pltpu.VMEM((4, D), jnp.float32),
            [pltpu.VMEM((1, B, D), jnp.float32) for _ in range(_NBUF)],
            [pltpu.SemaphoreType.DMA for _ in range(_NBUF)],
            [pltpu.SemaphoreType.DMA for _ in range(_NBUF)],
        ],
    )
    out_t = sc_add(xt, pe)
    return jnp.transpose(out_t, (1, 0, 2))
